# baseline (speedup 1.0000x reference)
import functools

import jax
import jax.numpy as jnp
import numpy as np
from jax import lax
from jax.experimental import pallas as pl
from jax.experimental.pallas import tpu as pltpu
from jax.experimental.pallas import tpu_sc as plsc

N = 10000
E = 160000
XD = 128
H = 8
D = 128
ZD = 64
ETOT = E + N
NC, NS, L = 2, 16, 16
NW = NC * NS
CH = 64
NCHUNK = 84
CH2 = 32
NCHUNK2 = 168
NB = 2
EPW = CH * NCHUNK
EPAD = EPW * NW
NPW = N // NS

_MESH = dict(core_axis_name="c", subcore_axis_name="s")

_PERM = np.array([32 * (p // 32) + 16 * (p % 2) + (p % 32) // 2
                  for p in range(H * D)], dtype=np.int32)


def _sc_pass1(elr, sd):
    @functools.partial(
        pl.kernel,
        out_type=(
            jax.ShapeDtypeStruct((NW * NCHUNK2, H, CH2), jnp.float32),
            jax.ShapeDtypeStruct((NC, N, 2 * H), jnp.float32),
        ),
        mesh=plsc.VectorSubcoreMesh(**_MESH),
        compiler_params=pltpu.CompilerParams(use_tc_tiling_on_sc=False, needs_layout_passes=False),
        scratch_types=(
            pltpu.VMEM((2, CH), jnp.int32),
            pltpu.VMEM((2, CH), jnp.int32),
            pltpu.VMEM((CH,), jnp.int32),
            pltpu.VMEM((CH,), jnp.int32),
            pltpu.VMEM((CH, 2 * H), jnp.float32),
            pltpu.VMEM((CH, 2 * H), jnp.float32),
            pltpu.VMEM((CH, 2 * H), jnp.float32),
            pltpu.VMEM((CH, 2 * H), jnp.float32),
            pltpu.VMEM((2, H, CH2), jnp.float32),
            pltpu.VMEM((2, H, CH2), jnp.float32),
            pltpu.VMEM((CH, 2 * H), jnp.float32),
            pltpu.VMEM((CH, 2 * H), jnp.float32),
            pltpu.VMEM((NPW, 2 * H), jnp.float32),
            pltpu.VMEM_SHARED((N, 2 * H), jnp.float32),
            pltpu.SemaphoreType.DMA,
            pltpu.SemaphoreType.DMA,
            pltpu.SemaphoreType.DMA,
            pltpu.SemaphoreType.DMA,
            pltpu.SemaphoreType.DMA,
            pltpu.SemaphoreType.DMA,
            pltpu.SemaphoreType.DMA,
            pltpu.SemaphoreType.DMA,
            pltpu.SemaphoreType.DMA,
            pltpu.SemaphoreType.DMA,
        ),
    )
    def k(elr_h, sd_h, ee_h, den_h, sdb0, sdb1, dsc0, dsc1, srows0, srows1,
          drows0, drows1, eehm0, eehm1, eeem0, eeem1, zbuf, den_sh,
          semsd0, semsd1, semgs0, semgs1, semgd0, semgd1,
          semw0, semw1, semdc0, semdc1):
        c = lax.axis_index("c")
        s = lax.axis_index("s")
        wid = s * NC + c
        lanes = lax.iota(jnp.int32, L)
        zero16 = jnp.zeros((L,), jnp.float32)
        sdb = (sdb0, sdb1)
        dsc = (dsc0, dsc1)
        srows = (srows0, srows1)
        drows = (drows0, drows1)
        eehm = (eehm0, eehm1)
        eeem = (eeem0, eeem1)
        semsd = (semsd0, semsd1)
        semgs = (semgs0, semgs1)
        semgd = (semgd0, semgd1)
        semw = (semw0, semw1)
        semdc = (semdc0, semdc1)

        @pl.loop(0, NPW)
        def _(r):
            zbuf[r, :] = zero16

        @pl.loop(0, CH)
        def _(r):
            eeem0[r, :] = zero16
            eeem1[r, :] = zero16

        pltpu.sync_copy(zbuf, den_sh.at[pl.ds(s * NPW, NPW)])
        plsc.subcore_barrier()

        def issue_smalls(g, p):
            base = wid * EPW + g * CH
            pltpu.async_copy(sd_h.at[:, pl.ds(base, CH)], sdb[p], semsd[p])

        def issue_gathers(p):
            pltpu.make_async_copy(sd_h.at[:, pl.ds(0, CH)], sdb[p],
                                  semsd[p]).wait()
            pltpu.async_copy(elr_h.at[sdb[p].at[0]], srows[p], semgs[p])
            pltpu.async_copy(elr_h.at[sdb[p].at[1]], drows[p], semgd[p])

        def phase(g, p, sw, pf):
            pltpu.make_async_copy(elr_h.at[sdb[p].at[0]], srows[p],
                                  semgs[p]).wait()
            pltpu.make_async_copy(elr_h.at[sdb[p].at[1]], drows[p],
                                  semgd[p]).wait()
            if sw:
                pltpu.make_async_copy(eehm[p], ee_h.at[pl.ds(0, 2)],
                                      semw[p]).wait()
                pltpu.make_async_copy(eeem[p], den_sh.at[dsc[p]],
                                      semdc[p]).wait()
            for gg in range(CH // L):
                row_idx = gg * L + lanes
                valid = (wid * EPW + g * CH + gg * L + lanes) < ETOT
                for h in range(H):
                    hvec = jnp.full((L,), h, jnp.int32)
                    elv = plsc.load_gather(srows[p], [row_idx, hvec])
                    erv = plsc.load_gather(drows[p], [row_idx, hvec + H])
                    e = elv + erv
                    e = jnp.where(e >= 0.0, e, 0.2 * e)
                    ee = jnp.where(valid, jnp.exp(e), 0.0)
                    eehm[p][gg // 2, h, pl.ds((gg % 2) * L, L)] = ee
                    plsc.store_scatter(eeem[p], [row_idx, hvec], ee)
            for q in range(CH // L):
                dsc[p][pl.ds(q * L, L)] = sdb[p][1, pl.ds(q * L, L)]
            if pf:
                issue_smalls(g + NB, p)
            pltpu.async_copy(eehm[p],
                             ee_h.at[pl.ds(wid * NCHUNK2 + 2 * g, 2)],
                             semw[p])
            pltpu.async_copy(eeem[p], den_sh.at[dsc[p]], semdc[p], add=True)
            if pf:
                issue_gathers(p)

        for p in range(NB):
            issue_smalls(p, p)
        for p in range(NB):
            issue_gathers(p)
        for p in range(NB):
            phase(p, p, sw=False, pf=True)

        @pl.loop(NB, NCHUNK - NB, step=NB)
        def _(i):
            for p in range(NB):
                phase(i + p, p, sw=True, pf=True)

        for p in range(NB):
            phase(NCHUNK - NB + p, p, sw=True, pf=False)
        for p in range(NB):
            pltpu.make_async_copy(eehm[p], ee_h.at[pl.ds(0, 2)],
                                  semw[p]).wait()
            pltpu.make_async_copy(eeem[p], den_sh.at[dsc[p]],
                                  semdc[p]).wait()

        plsc.subcore_barrier()
        pltpu.sync_copy(den_sh.at[pl.ds(s * NPW, NPW)],
                        den_h.at[c, pl.ds(s * NPW, NPW)])

    return k(elr, sd)


def _sc_pass2(feat, ee, dent, sd):
    @functools.partial(
        pl.kernel,
        out_type=jax.ShapeDtypeStruct((NC, N, D), jnp.float32),
        mesh=plsc.VectorSubcoreMesh(**_MESH),
        compiler_params=pltpu.CompilerParams(use_tc_tiling_on_sc=False, needs_layout_passes=False),
        scratch_types=(
            pltpu.VMEM((2, CH2), jnp.int32),
            pltpu.VMEM((2, CH2), jnp.int32),
            pltpu.VMEM((CH2,), jnp.int32),
            pltpu.VMEM((CH2,), jnp.int32),
            pltpu.VMEM((CH2, H, D), jnp.bfloat16),
            pltpu.VMEM((CH2, H, D), jnp.bfloat16),
            pltpu.VMEM((CH2, 4 * H), jnp.float32),
            pltpu.VMEM((CH2, 4 * H), jnp.float32),
            pltpu.VMEM((H, CH2), jnp.float32),
            pltpu.VMEM((H, CH2), jnp.float32),
            pltpu.VMEM((H, CH2), jnp.float32),
            pltpu.VMEM((CH2, D), jnp.float32),
            pltpu.VMEM((CH2, D), jnp.float32),
            pltpu.VMEM_SHARED((N, D), jnp.float32),
            pltpu.SemaphoreType.DMA,
            pltpu.SemaphoreType.DMA,
            pltpu.SemaphoreType.DMA,
            pltpu.SemaphoreType.DMA,
            pltpu.SemaphoreType.DMA,
            pltpu.SemaphoreType.DMA,
            pltpu.SemaphoreType.DMA,
            pltpu.SemaphoreType.DMA,
            pltpu.SemaphoreType.DMA,
            pltpu.SemaphoreType.DMA,
        ),
    )
    def k(feat_h, ee_h, dent_h, sd_h, out_h, sdb0, sdb1, msc0, msc1,
          featb0, featb1, rrows0, rrows1, eeb0, eeb1, ab, msgb0, msgb1,
          out_sh, semsd0, semsd1, semee0, semee1, semr0, semr1,
          semf0, semf1, semsc0, semsc1):
        c = lax.axis_index("c")
        s = lax.axis_index("s")
        wid = s * NC + c
        lanes = lax.iota(jnp.int32, L)
        zero16 = jnp.zeros((L,), jnp.float32)
        sdb = (sdb0, sdb1)
        msc = (msc0, msc1)
        featb = (featb0, featb1)
        rrows = (rrows0, rrows1)
        eeb = (eeb0, eeb1)
        msgb = (msgb0, msgb1)
        semsd = (semsd0, semsd1)
        semee = (semee0, semee1)
        semr = (semr0, semr1)
        semf = (semf0, semf1)
        semsc = (semsc0, semsc1)

        @pl.loop(0, CH2)
        def _(r):
            for cb in range(D // L):
                msgb0[r, pl.ds(cb * L, L)] = zero16

        for t in range(NPW // CH2):
            pltpu.sync_copy(msgb0, out_sh.at[pl.ds(s * NPW + t * CH2, CH2)])
        rem = NPW % CH2
        pltpu.sync_copy(msgb0.at[pl.ds(0, rem)],
                        out_sh.at[pl.ds(s * NPW + (NPW // CH2) * CH2, rem)])
        plsc.subcore_barrier()

        def issue_smalls(g, p):
            base = wid * EPW + g * CH2
            pltpu.async_copy(sd_h.at[:, pl.ds(base, CH2)], sdb[p], semsd[p])
            pltpu.async_copy(ee_h.at[wid * NCHUNK2 + g], eeb[p], semee[p])

        def issue_gathers(p):
            pltpu.make_async_copy(sd_h.at[:, pl.ds(0, CH2)], sdb[p],
                                  semsd[p]).wait()
            pltpu.async_copy(dent_h.at[sdb[p].at[1]], rrows[p], semr[p])
            pltpu.async_copy(feat_h.at[sdb[p].at[0]], featb[p], semf[p])

        def phase(g, p, sw, pf):
            pltpu.make_async_copy(dent_h.at[sdb[p].at[1]], rrows[p],
                                  semr[p]).wait()
            pltpu.make_async_copy(ee_h.at[wid * NCHUNK2 + g], eeb[p],
                                  semee[p]).wait()
            for gg in range(CH2 // L):
                row_idx = gg * L + lanes
                for h in range(H):
                    hvec = jnp.full((L,), h, jnp.int32)
                    rd0 = plsc.load_gather(rrows[p], [row_idx, hvec])
                    rd1 = plsc.load_gather(rrows[p], [row_idx, hvec + 2 * H])
                    ab[h, pl.ds(gg * L, L)] = (
                        eeb[p][h, pl.ds(gg * L, L)] / (rd0 + rd1))
            pltpu.make_async_copy(feat_h.at[sdb[p].at[0]], featb[p],
                                  semf[p]).wait()
            if sw:
                pltpu.make_async_copy(msgb[p], out_sh.at[msc[p]],
                                      semsc[p]).wait()
            msc[p][pl.ds(0, L)] = sdb[p][1, pl.ds(0, L)]
            msc[p][pl.ds(L, L)] = sdb[p][1, pl.ds(L, L)]
            if pf:
                issue_smalls(g + NB, p)

            @pl.loop(0, CH2)
            def _(j):
                jvec = jnp.full((L,), j, jnp.int32)
                avs = [plsc.load_gather(
                    ab, [jnp.full((L,), h, jnp.int32), jvec])
                    for h in range(H)]
                fb = featb[p]
                mb = msgb[p]
                for cb2 in range(D // (2 * L)):
                    va, vb = plsc.unpack(fb[j, 0, pl.ds(cb2 * 2 * L, 2 * L)],
                                         format=plsc.PackFormat.INTERLEAVED)
                    acc_a = avs[0] * va
                    acc_b = avs[0] * vb
                    for h in range(1, H):
                        va, vb = plsc.unpack(
                            fb[j, h, pl.ds(cb2 * 2 * L, 2 * L)],
                            format=plsc.PackFormat.INTERLEAVED)
                        acc_a = acc_a + avs[h] * va
                        acc_b = acc_b + avs[h] * vb
                    mb[j, pl.ds(cb2 * 2 * L, L)] = acc_a
                    mb[j, pl.ds(cb2 * 2 * L + L, L)] = acc_b

            pltpu.async_copy(msgb[p], out_sh.at[msc[p]], semsc[p], add=True)
            if pf:
                issue_gathers(p)

        for p in range(NB):
            issue_smalls(p, p)
        for p in range(NB):
            issue_gathers(p)
        for p in range(NB):
            phase(p, p, sw=False, pf=True)

        @pl.loop(NB, NCHUNK2 - NB, step=NB)
        def _(i):
            for p in range(NB):
                phase(i + p, p, sw=True, pf=True)

        for p in range(NB):
            phase(NCHUNK2 - NB + p, p, sw=True, pf=False)
        for p in range(NB):
            pltpu.make_async_copy(msgb[p], out_sh.at[msc[p]], semsc[p]).wait()

        plsc.subcore_barrier()
        pltpu.sync_copy(out_sh.at[pl.ds(s * NPW, NPW)],
                        out_h.at[c, pl.ds(s * NPW, NPW)])

    return k(feat, ee, dent, sd)


def _tc1_body(x_ref, wfc_ref, a2_ref, wres_ref, feat_ref, elr_ref, res_ref):
    xb = x_ref[...]
    f = jnp.dot(xb, wfc_ref[...], preferred_element_type=jnp.float32)
    feat_ref[...] = f.astype(jnp.bfloat16)
    elr_ref[...] = jnp.dot(f, a2_ref[...], preferred_element_type=jnp.float32)
    res_ref[...] = jnp.dot(xb, wres_ref[...], preferred_element_type=jnp.float32)


def _tc3_body(op_ref, res_ref, b_ref, g_ref, be_ref, wmu_ref, bmu_ref,
              wls_ref, bls_ref, mu_ref, ls_ref):
    h = op_ref[0] + op_ref[1] + res_ref[...] + b_ref[...]
    h = jnp.maximum(h, 0.0)
    mean = jnp.mean(h, axis=0, keepdims=True)
    var = jnp.mean((h - mean) ** 2, axis=0, keepdims=True)
    hn = (h - mean) * lax.rsqrt(var + 1e-5) * g_ref[...] + be_ref[...]
    mu_ref[...] = jnp.dot(hn, wmu_ref[...],
                          preferred_element_type=jnp.float32) + bmu_ref[...]
    ls = jnp.dot(hn, wls_ref[...],
                 preferred_element_type=jnp.float32) + bls_ref[...]
    ls_ref[...] = jnp.clip(ls, -10.0, 10.0)


def kernel(x, edge_index, W_fc, attn_l, attn_r, W_res, bias_gat, gamma,
           beta, W_mu, b_mu, W_ls, b_ls):
    n = x.shape[0]
    loops = jnp.arange(n, dtype=edge_index.dtype)
    pad = jnp.zeros((EPAD - ETOT,), edge_index.dtype)
    src = jnp.concatenate([edge_index[0], loops, pad])
    dst = jnp.concatenate([edge_index[1], loops, pad])
    sd = jnp.stack([src, dst])

    eye = jnp.eye(H, dtype=jnp.float32)
    A2 = jnp.concatenate(
        [(attn_l[:, :, None] * eye[:, None, :]).reshape(H * D, H),
         (attn_r[:, :, None] * eye[:, None, :]).reshape(H * D, H)], axis=1)
    W_fc = W_fc[:, _PERM]
    A2 = A2[_PERM]
    W_res_sum = W_res.reshape(XD, H, D).sum(axis=1)
    bias_sum = bias_gat.reshape(H, D).sum(axis=0).reshape(1, D)

    R = 2000
    feat, elr, resid = pl.pallas_call(
        _tc1_body,
        grid=(n // R,),
        in_specs=[
            pl.BlockSpec((R, XD), lambda i: (i, 0)),
            pl.BlockSpec((XD, H * D), lambda i: (0, 0)),
            pl.BlockSpec((H * D, 2 * H), lambda i: (0, 0)),
            pl.BlockSpec((XD, D), lambda i: (0, 0)),
        ],
        out_specs=[
            pl.BlockSpec((R, H * D), lambda i: (i, 0)),
            pl.BlockSpec((R, 2 * H), lambda i: (i, 0)),
            pl.BlockSpec((R, D), lambda i: (i, 0)),
        ],
        out_shape=[
            jax.ShapeDtypeStruct((n, H * D), jnp.bfloat16),
            jax.ShapeDtypeStruct((n, 2 * H), jnp.float32),
            jax.ShapeDtypeStruct((n, D), jnp.float32),
        ],
    )(x, W_fc, A2, W_res_sum)
    feat = feat.reshape(n, H, D)

    ee, denom = _sc_pass1(elr, sd)
    dent = jnp.transpose(denom, (1, 0, 2)).reshape(N, 2 * NC * H)
    outp = _sc_pass2(feat, ee, dent, sd)

    mu, log_sigma = pl.pallas_call(
        _tc3_body,
        out_shape=[
            jax.ShapeDtypeStruct((n, ZD), jnp.float32),
            jax.ShapeDtypeStruct((n, ZD), jnp.float32),
        ],
    )(outp, resid, bias_sum, gamma.reshape(1, D), beta.reshape(1, D),
      W_mu, b_mu.reshape(1, ZD), W_ls, b_ls.reshape(1, ZD))
    return (mu, log_sigma)


# speedup vs baseline: 1.1045x; 1.1045x over previous
import functools

import jax
import jax.numpy as jnp
import numpy as np
from jax import lax
from jax.experimental import pallas as pl
from jax.experimental.pallas import tpu as pltpu
from jax.experimental.pallas import tpu_sc as plsc

N = 10000
E = 160000
XD = 128
H = 8
D = 128
ZD = 64
ETOT = E + N
NC, NS, L = 2, 16, 16
NW = NC * NS
CH = 64
NCHUNK = 84
CH2 = 32
NCHUNK2 = 168
NB = 2
EPW = CH * NCHUNK
EPAD = EPW * NW
NPW = N // NS

_MESH = dict(core_axis_name="c", subcore_axis_name="s")

_PERM = np.array([32 * (p // 32) + 16 * (p % 2) + (p % 32) // 2
                  for p in range(H * D)], dtype=np.int32)


def _sc_pass1(elr, sd):
    @functools.partial(
        pl.kernel,
        out_type=(
            jax.ShapeDtypeStruct((NW * NCHUNK2, H, CH2), jnp.float32),
            jax.ShapeDtypeStruct((NC, N, 2 * H), jnp.float32),
        ),
        mesh=plsc.VectorSubcoreMesh(**_MESH),
        compiler_params=pltpu.CompilerParams(use_tc_tiling_on_sc=False, needs_layout_passes=False),
        scratch_types=(
            pltpu.VMEM((2, CH), jnp.int32),
            pltpu.VMEM((2, CH), jnp.int32),
            pltpu.VMEM((CH,), jnp.int32),
            pltpu.VMEM((CH,), jnp.int32),
            pltpu.VMEM((CH, 2 * H), jnp.float32),
            pltpu.VMEM((CH, 2 * H), jnp.float32),
            pltpu.VMEM((CH, 2 * H), jnp.float32),
            pltpu.VMEM((CH, 2 * H), jnp.float32),
            pltpu.VMEM((2, H, CH2), jnp.float32),
            pltpu.VMEM((2, H, CH2), jnp.float32),
            pltpu.VMEM((CH, 2 * H), jnp.float32),
            pltpu.VMEM((CH, 2 * H), jnp.float32),
            pltpu.VMEM((NPW, 2 * H), jnp.float32),
            pltpu.VMEM_SHARED((N, 2 * H), jnp.float32),
            pltpu.SemaphoreType.DMA,
            pltpu.SemaphoreType.DMA,
            pltpu.SemaphoreType.DMA,
            pltpu.SemaphoreType.DMA,
            pltpu.SemaphoreType.DMA,
            pltpu.SemaphoreType.DMA,
            pltpu.SemaphoreType.DMA,
            pltpu.SemaphoreType.DMA,
            pltpu.SemaphoreType.DMA,
            pltpu.SemaphoreType.DMA,
        ),
    )
    def k(elr_h, sd_h, ee_h, den_h, sdb0, sdb1, dsc0, dsc1, srows0, srows1,
          drows0, drows1, eehm0, eehm1, eeem0, eeem1, zbuf, den_sh,
          semsd0, semsd1, semgs0, semgs1, semgd0, semgd1,
          semw0, semw1, semdc0, semdc1):
        c = lax.axis_index("c")
        s = lax.axis_index("s")
        wid = s * NC + c
        lanes = lax.iota(jnp.int32, L)
        zero16 = jnp.zeros((L,), jnp.float32)
        sdb = (sdb0, sdb1)
        dsc = (dsc0, dsc1)
        srows = (srows0, srows1)
        drows = (drows0, drows1)
        eehm = (eehm0, eehm1)
        eeem = (eeem0, eeem1)
        semsd = (semsd0, semsd1)
        semgs = (semgs0, semgs1)
        semgd = (semgd0, semgd1)
        semw = (semw0, semw1)
        semdc = (semdc0, semdc1)

        @pl.loop(0, NPW)
        def _(r):
            zbuf[r, :] = zero16

        @pl.loop(0, CH)
        def _(r):
            eeem0[r, :] = zero16
            eeem1[r, :] = zero16

        pltpu.sync_copy(zbuf, den_sh.at[pl.ds(s * NPW, NPW)])
        plsc.subcore_barrier()

        def issue_smalls(g, p):
            base = wid * EPW + g * CH
            pltpu.async_copy(sd_h.at[:, pl.ds(base, CH)], sdb[p], semsd[p])

        def issue_gathers(p):
            pltpu.make_async_copy(sd_h.at[:, pl.ds(0, CH)], sdb[p],
                                  semsd[p]).wait()
            pltpu.async_copy(elr_h.at[sdb[p].at[0]], srows[p], semgs[p])
            pltpu.async_copy(elr_h.at[sdb[p].at[1]], drows[p], semgd[p])

        def phase(g, p, sw, pf):
            pltpu.make_async_copy(elr_h.at[sdb[p].at[0]], srows[p],
                                  semgs[p]).wait()
            pltpu.make_async_copy(elr_h.at[sdb[p].at[1]], drows[p],
                                  semgd[p]).wait()
            if sw:
                pltpu.make_async_copy(eehm[p], ee_h.at[pl.ds(0, 2)],
                                      semw[p]).wait()
                pltpu.make_async_copy(eeem[p], den_sh.at[dsc[p]],
                                      semdc[p]).wait()
            for gg in range(CH // L):
                row_idx = gg * L + lanes
                valid = (wid * EPW + g * CH + gg * L + lanes) < ETOT
                for h in range(H):
                    hvec = jnp.full((L,), h, jnp.int32)
                    elv = plsc.load_gather(srows[p], [row_idx, hvec])
                    erv = plsc.load_gather(drows[p], [row_idx, hvec + H])
                    e = elv + erv
                    e = jnp.where(e >= 0.0, e, 0.2 * e)
                    ee = jnp.where(valid, jnp.exp(e), 0.0)
                    eehm[p][gg // 2, h, pl.ds((gg % 2) * L, L)] = ee
                    plsc.store_scatter(eeem[p], [row_idx, hvec], ee)
            for q in range(CH // L):
                dsc[p][pl.ds(q * L, L)] = sdb[p][1, pl.ds(q * L, L)]
            if pf:
                issue_smalls(g + NB, p)
            pltpu.async_copy(eehm[p],
                             ee_h.at[pl.ds(wid * NCHUNK2 + 2 * g, 2)],
                             semw[p])
            pltpu.async_copy(eeem[p], den_sh.at[dsc[p]], semdc[p], add=True)
            if pf:
                issue_gathers(p)

        for p in range(NB):
            issue_smalls(p, p)
        for p in range(NB):
            issue_gathers(p)
        for p in range(NB):
            phase(p, p, sw=False, pf=True)

        @pl.loop(NB, NCHUNK - NB, step=NB)
        def _(i):
            for p in range(NB):
                phase(i + p, p, sw=True, pf=True)

        for p in range(NB):
            phase(NCHUNK - NB + p, p, sw=True, pf=False)
        for p in range(NB):
            pltpu.make_async_copy(eehm[p], ee_h.at[pl.ds(0, 2)],
                                  semw[p]).wait()
            pltpu.make_async_copy(eeem[p], den_sh.at[dsc[p]],
                                  semdc[p]).wait()

        plsc.subcore_barrier()
        pltpu.sync_copy(den_sh.at[pl.ds(s * NPW, NPW)],
                        den_h.at[c, pl.ds(s * NPW, NPW)])

    return k(elr, sd)


def _sc_pass2(feat, ee, rden, sd):
    @functools.partial(
        pl.kernel,
        out_type=jax.ShapeDtypeStruct((NC, N, D), jnp.float32),
        mesh=plsc.VectorSubcoreMesh(**_MESH),
        compiler_params=pltpu.CompilerParams(use_tc_tiling_on_sc=False, needs_layout_passes=False),
        scratch_types=(
            pltpu.VMEM((2, CH2), jnp.int32),
            pltpu.VMEM((2, CH2), jnp.int32),
            pltpu.VMEM((CH2,), jnp.int32),
            pltpu.VMEM((CH2,), jnp.int32),
            pltpu.VMEM((CH2, H, D), jnp.bfloat16),
            pltpu.VMEM((CH2, H, D), jnp.bfloat16),
            pltpu.VMEM((CH2, 2 * H), jnp.float32),
            pltpu.VMEM((CH2, 2 * H), jnp.float32),
            pltpu.VMEM((H, CH2), jnp.float32),
            pltpu.VMEM((H, CH2), jnp.float32),
            pltpu.VMEM((H, CH2), jnp.float32),
            pltpu.VMEM((CH2, D), jnp.float32),
            pltpu.VMEM((CH2, D), jnp.float32),
            pltpu.VMEM_SHARED((N, D), jnp.float32),
            pltpu.SemaphoreType.DMA,
            pltpu.SemaphoreType.DMA,
            pltpu.SemaphoreType.DMA,
            pltpu.SemaphoreType.DMA,
            pltpu.SemaphoreType.DMA,
            pltpu.SemaphoreType.DMA,
            pltpu.SemaphoreType.DMA,
            pltpu.SemaphoreType.DMA,
            pltpu.SemaphoreType.DMA,
            pltpu.SemaphoreType.DMA,
        ),
    )
    def k(feat_h, ee_h, rden_h, sd_h, out_h, sdb0, sdb1, msc0, msc1,
          featb0, featb1, rrows0, rrows1, eeb0, eeb1, ab, msgb0, msgb1,
          out_sh, semsd0, semsd1, semee0, semee1, semr0, semr1,
          semf0, semf1, semsc0, semsc1):
        c = lax.axis_index("c")
        s = lax.axis_index("s")
        wid = s * NC + c
        lanes = lax.iota(jnp.int32, L)
        zero16 = jnp.zeros((L,), jnp.float32)
        sdb = (sdb0, sdb1)
        msc = (msc0, msc1)
        featb = (featb0, featb1)
        rrows = (rrows0, rrows1)
        eeb = (eeb0, eeb1)
        msgb = (msgb0, msgb1)
        semsd = (semsd0, semsd1)
        semee = (semee0, semee1)
        semr = (semr0, semr1)
        semf = (semf0, semf1)
        semsc = (semsc0, semsc1)

        @pl.loop(0, CH2)
        def _(r):
            for cb in range(D // L):
                msgb0[r, pl.ds(cb * L, L)] = zero16

        for t in range(NPW // CH2):
            pltpu.sync_copy(msgb0, out_sh.at[pl.ds(s * NPW + t * CH2, CH2)])
        rem = NPW % CH2
        pltpu.sync_copy(msgb0.at[pl.ds(0, rem)],
                        out_sh.at[pl.ds(s * NPW + (NPW // CH2) * CH2, rem)])
        plsc.subcore_barrier()

        def issue_smalls(g, p):
            base = wid * EPW + g * CH2
            pltpu.async_copy(sd_h.at[:, pl.ds(base, CH2)], sdb[p], semsd[p])
            pltpu.async_copy(ee_h.at[wid * NCHUNK2 + g], eeb[p], semee[p])

        def issue_gathers(p):
            pltpu.make_async_copy(sd_h.at[:, pl.ds(0, CH2)], sdb[p],
                                  semsd[p]).wait()
            pltpu.async_copy(rden_h.at[sdb[p].at[1]], rrows[p], semr[p])
            pltpu.async_copy(feat_h.at[sdb[p].at[0]], featb[p], semf[p])

        def phase(g, p, sw, pf):
            pltpu.make_async_copy(rden_h.at[sdb[p].at[1]], rrows[p],
                                  semr[p]).wait()
            pltpu.make_async_copy(ee_h.at[wid * NCHUNK2 + g], eeb[p],
                                  semee[p]).wait()
            for gg in range(CH2 // L):
                row_idx = gg * L + lanes
                for h in range(H):
                    hvec = jnp.full((L,), h, jnp.int32)
                    rd = plsc.load_gather(rrows[p], [row_idx, hvec])
                    ab[h, pl.ds(gg * L, L)] = eeb[p][h, pl.ds(gg * L, L)] * rd
            pltpu.make_async_copy(feat_h.at[sdb[p].at[0]], featb[p],
                                  semf[p]).wait()
            if sw:
                pltpu.make_async_copy(msgb[p], out_sh.at[msc[p]],
                                      semsc[p]).wait()
            msc[p][pl.ds(0, L)] = sdb[p][1, pl.ds(0, L)]
            msc[p][pl.ds(L, L)] = sdb[p][1, pl.ds(L, L)]
            if pf:
                issue_smalls(g + NB, p)

            @pl.loop(0, CH2)
            def _(j):
                jvec = jnp.full((L,), j, jnp.int32)
                avs = [plsc.load_gather(
                    ab, [jnp.full((L,), h, jnp.int32), jvec])
                    for h in range(H)]
                fb = featb[p]
                mb = msgb[p]
                for cb2 in range(D // (2 * L)):
                    va, vb = plsc.unpack(fb[j, 0, pl.ds(cb2 * 2 * L, 2 * L)],
                                         format=plsc.PackFormat.INTERLEAVED)
                    acc_a = avs[0] * va
                    acc_b = avs[0] * vb
                    for h in range(1, H):
                        va, vb = plsc.unpack(
                            fb[j, h, pl.ds(cb2 * 2 * L, 2 * L)],
                            format=plsc.PackFormat.INTERLEAVED)
                        acc_a = acc_a + avs[h] * va
                        acc_b = acc_b + avs[h] * vb
                    mb[j, pl.ds(cb2 * 2 * L, L)] = acc_a
                    mb[j, pl.ds(cb2 * 2 * L + L, L)] = acc_b

            pltpu.async_copy(msgb[p], out_sh.at[msc[p]], semsc[p], add=True)
            if pf:
                issue_gathers(p)

        for p in range(NB):
            issue_smalls(p, p)
        for p in range(NB):
            issue_gathers(p)
        for p in range(NB):
            phase(p, p, sw=False, pf=True)

        @pl.loop(NB, NCHUNK2 - NB, step=NB)
        def _(i):
            for p in range(NB):
                phase(i + p, p, sw=True, pf=True)

        for p in range(NB):
            phase(NCHUNK2 - NB + p, p, sw=True, pf=False)
        for p in range(NB):
            pltpu.make_async_copy(msgb[p], out_sh.at[msc[p]], semsc[p]).wait()

        plsc.subcore_barrier()
        pltpu.sync_copy(out_sh.at[pl.ds(s * NPW, NPW)],
                        out_h.at[c, pl.ds(s * NPW, NPW)])

    return k(feat, ee, rden, sd)


def _tc1_body(x_ref, wfc_ref, a2_ref, wres_ref, feat_ref, elr_ref, res_ref):
    xb = x_ref[...]
    f = jnp.dot(xb, wfc_ref[...], preferred_element_type=jnp.float32)
    feat_ref[...] = f.astype(jnp.bfloat16)
    elr_ref[...] = jnp.dot(f, a2_ref[...], preferred_element_type=jnp.float32)
    res_ref[...] = jnp.dot(xb, wres_ref[...], preferred_element_type=jnp.float32)


def _tc2_body(d_ref, o_ref):
    o_ref[...] = 1.0 / (d_ref[0] + d_ref[1])


def _tc3_body(op_ref, res_ref, b_ref, g_ref, be_ref, wmu_ref, bmu_ref,
              wls_ref, bls_ref, mu_ref, ls_ref):
    h = op_ref[0] + op_ref[1] + res_ref[...] + b_ref[...]
    h = jnp.maximum(h, 0.0)
    mean = jnp.mean(h, axis=0, keepdims=True)
    var = jnp.mean((h - mean) ** 2, axis=0, keepdims=True)
    hn = (h - mean) * lax.rsqrt(var + 1e-5) * g_ref[...] + be_ref[...]
    mu_ref[...] = jnp.dot(hn, wmu_ref[...],
                          preferred_element_type=jnp.float32) + bmu_ref[...]
    ls = jnp.dot(hn, wls_ref[...],
                 preferred_element_type=jnp.float32) + bls_ref[...]
    ls_ref[...] = jnp.clip(ls, -10.0, 10.0)


def kernel(x, edge_index, W_fc, attn_l, attn_r, W_res, bias_gat, gamma,
           beta, W_mu, b_mu, W_ls, b_ls):
    n = x.shape[0]
    loops = jnp.arange(n, dtype=edge_index.dtype)
    pad = jnp.zeros((EPAD - ETOT,), edge_index.dtype)
    src = jnp.concatenate([edge_index[0], loops, pad])
    dst = jnp.concatenate([edge_index[1], loops, pad])
    sd = jnp.stack([src, dst])

    eye = jnp.eye(H, dtype=jnp.float32)
    A2 = jnp.concatenate(
        [(attn_l[:, :, None] * eye[:, None, :]).reshape(H * D, H),
         (attn_r[:, :, None] * eye[:, None, :]).reshape(H * D, H)], axis=1)
    W_fc = W_fc[:, _PERM]
    A2 = A2[_PERM]
    W_res_sum = W_res.reshape(XD, H, D).sum(axis=1)
    bias_sum = bias_gat.reshape(H, D).sum(axis=0).reshape(1, D)

    R = 2000
    feat, elr, resid = pl.pallas_call(
        _tc1_body,
        grid=(n // R,),
        in_specs=[
            pl.BlockSpec((R, XD), lambda i: (i, 0)),
            pl.BlockSpec((XD, H * D), lambda i: (0, 0)),
            pl.BlockSpec((H * D, 2 * H), lambda i: (0, 0)),
            pl.BlockSpec((XD, D), lambda i: (0, 0)),
        ],
        out_specs=[
            pl.BlockSpec((R, H * D), lambda i: (i, 0)),
            pl.BlockSpec((R, 2 * H), lambda i: (i, 0)),
            pl.BlockSpec((R, D), lambda i: (i, 0)),
        ],
        out_shape=[
            jax.ShapeDtypeStruct((n, H * D), jnp.bfloat16),
            jax.ShapeDtypeStruct((n, 2 * H), jnp.float32),
            jax.ShapeDtypeStruct((n, D), jnp.float32),
        ],
    )(x, W_fc, A2, W_res_sum)
    feat = feat.reshape(n, H, D)

    ee, denom = _sc_pass1(elr, sd)

    d2 = denom.reshape(NC, N * 2 * H // D, D)
    rden = pl.pallas_call(
        _tc2_body,
        out_shape=jax.ShapeDtypeStruct((N * 2 * H // D, D), jnp.float32),
    )(d2).reshape(N, 2 * H)

    outp = _sc_pass2(feat, ee, rden, sd)

    mu, log_sigma = pl.pallas_call(
        _tc3_body,
        out_shape=[
            jax.ShapeDtypeStruct((n, ZD), jnp.float32),
            jax.ShapeDtypeStruct((n, ZD), jnp.float32),
        ],
    )(outp, resid, bias_sum, gamma.reshape(1, D), beta.reshape(1, D),
      W_mu, b_mu.reshape(1, ZD), W_ls, b_ls.reshape(1, ZD))
    return (mu, log_sigma)
